# baseline (speedup 1.0000x reference)
import functools

import jax
import jax.numpy as jnp
from jax import lax
from jax.experimental import pallas as pl
from jax.experimental.pallas import tpu as pltpu
from jax.experimental.pallas import tpu_sc as plsc

B, S, D = 8, 512, 768
SPAN_NUM = 2048
MAX_W = 4
D_FF = 3072
N_CLS = 3

M_TILE = 1024
NG = B * SPAN_NUM // M_TILE
LANES = 128
LN_EPS = 1e-12

NW = 32
SP_W = B * SPAN_NUM // NW
CH = 16
N_IT = SP_W // CH
FCH = D // 16


def _sc_gather(x_hbm, idx_hbm, out_hbm, idx_v, rows_v, out_v, sem):
    wid = lax.axis_index("s") * 2 + lax.axis_index("c")
    base = wid * SP_W

    def body(it, carry):
        sp0 = pl.multiple_of(base + it * CH, CH)
        pltpu.sync_copy(idx_hbm.at[pl.ds(sp0 * MAX_W, CH * MAX_W)], idx_v)
        pltpu.async_copy(x_hbm.at[idx_v], rows_v, sem).wait()

        def fbody(cc, c2):
            sl = pl.ds(cc * 16, 16)
            for i in range(CH):
                out_v[i, sl] = (rows_v[4 * i, sl] + rows_v[4 * i + 1, sl]
                                + rows_v[4 * i + 2, sl]
                                + rows_v[4 * i + 3, sl])
            return c2

        lax.fori_loop(0, FCH, fbody, 0)
        pltpu.sync_copy(out_v, out_hbm.at[pl.ds(sp0, CH), :])
        return carry

    lax.fori_loop(0, N_IT, body, 0)


@jax.jit
def _gather(x_rows, idx):
    mesh = plsc.VectorSubcoreMesh(core_axis_name="c", subcore_axis_name="s")
    return pl.kernel(
        _sc_gather,
        mesh=mesh,
        out_type=jax.ShapeDtypeStruct((B * SPAN_NUM, D), jnp.float32),
        scratch_types=[
            pltpu.VMEM((CH * MAX_W,), jnp.int32),
            pltpu.VMEM((CH * MAX_W, D), jnp.float32),
            pltpu.VMEM((CH, D), jnp.float32),
            pltpu.SemaphoreType.DMA,
        ],
    )(x_rows, idx)


def _fused_body(e_ref, iv_ref, wi_f, bi_f, wo_f, bo_f,
                wi_r, bi_r, wo_r, bo_r, wg, u, cb, out_ref):
    e = e_ref[...] * iv_ref[0]
    e_bf = e.astype(jnp.bfloat16)

    def decoder(wi, bi, wo, bo):
        nc = D_FF // D
        acc = None
        for k in range(nc):
            t = (jnp.dot(e_bf, wi[:, k * D:(k + 1) * D],
                         preferred_element_type=jnp.float32)
                 + bi[:, k * D:(k + 1) * D])
            h = (t + t * jax.lax.erf(t)).astype(jnp.bfloat16)
            part = jnp.dot(h, wo[k * D:(k + 1) * D, :],
                           preferred_element_type=jnp.float32)
            acc = part if acc is None else acc + part
        y = acc + bo[...] + e
        m = jnp.mean(y, axis=-1, keepdims=True)
        s2 = jnp.mean(y * y, axis=-1, keepdims=True)
        inv = jax.lax.rsqrt(jnp.maximum(s2 - m * m, 0.0) + LN_EPS)
        return y.astype(jnp.bfloat16), m, inv

    y_f, m_f, i_f = decoder(wi_f, bi_f, wo_f, bo_f)
    y_r, m_r, i_r = decoder(wi_r, bi_r, wo_r, bo_r)

    z_f = (jnp.dot(y_f, wg[0], preferred_element_type=jnp.float32)
           - m_f * u[0]) * i_f
    z_r = (jnp.dot(y_r, wg[1], preferred_element_type=jnp.float32)
           - m_r * u[1]) * i_r
    out_ref[...] = z_f + z_r + cb[...]


@jax.jit
def _fused(e, iv, wi_f, bi_f, wo_f, bo_f,
           wi_r, bi_r, wo_r, bo_r, wg, u, cb):
    full = lambda shape: pl.BlockSpec(shape, lambda i: (0,) * len(shape))
    return pl.pallas_call(
        _fused_body,
        grid=(NG,),
        in_specs=[
            pl.BlockSpec((M_TILE, D), lambda i: (i, 0)),
            pl.BlockSpec((1, M_TILE, 1), lambda i: (i, 0, 0)),
            full((D, D_FF)), full((1, D_FF)), full((D_FF, D)), full((1, D)),
            full((D, D_FF)), full((1, D_FF)), full((D_FF, D)), full((1, D)),
            full((2, D, LANES)), full((2, 1, LANES)), full((1, LANES)),
        ],
        out_specs=pl.BlockSpec((M_TILE, LANES), lambda i: (i, 0)),
        out_shape=jax.ShapeDtypeStruct((B * SPAN_NUM, LANES), jnp.float32),
        compiler_params=pltpu.CompilerParams(
            dimension_semantics=("parallel",)),
    )(e, iv, wi_f, bi_f, wo_f, bo_f, wi_r, bi_r, wo_r, bo_r, wg, u, cb)


def kernel(input_bert_features, attention_mask, spans, span_mask,
           related_spans_tensor, sentence_length, Wi_f, bi_f, Wo_f, bo_f,
           g_f, be_f, Wi_r, bi_r, Wo_r, bo_r, g_r, be_r, Wa, ba, Wop, bop):
    start = spans[..., 0]
    width = spans[..., 2]
    boff = (jnp.arange(B, dtype=jnp.int32) * S)[:, None, None]
    offs = jnp.arange(MAX_W, dtype=jnp.int32)[None, None, :]
    idx = boff + start[..., None] + offs
    idx = jnp.where(offs < width[..., None], idx, B * S)
    idx = idx.reshape(B * SPAN_NUM * MAX_W)
    invw = (span_mask.astype(jnp.float32)
            / jnp.maximum(width, 1).astype(jnp.float32)).reshape(-1)
    x_rows = jnp.concatenate(
        [input_bert_features.reshape(B * S, D),
         jnp.zeros((8, D), jnp.float32)], axis=0)

    e = _gather(x_rows, idx)
    iv = invw.reshape(NG, M_TILE, 1)

    wc = jnp.zeros((2, D, LANES), jnp.float32)
    wc = wc.at[0, :, :N_CLS].set(Wa).at[1, :, N_CLS:2 * N_CLS].set(Wop)
    wg = wc * jnp.stack([g_f, g_r])[:, :, None]
    u = jnp.sum(wg, axis=1, keepdims=True)
    cb = (be_f @ wc[0] + be_r @ wc[1]).reshape(1, LANES)
    cb = cb.at[0, :N_CLS].add(ba).at[0, N_CLS:2 * N_CLS].add(bop)

    bf = jnp.bfloat16
    c = 0.7071067811865476
    out = _fused(e, iv,
                 (Wi_f * c).astype(bf), (bi_f * c).reshape(1, D_FF),
                 (Wo_f * c).astype(bf), bo_f.reshape(1, D),
                 (Wi_r * c).astype(bf), (bi_r * c).reshape(1, D_FF),
                 (Wo_r * c).astype(bf), bo_r.reshape(1, D),
                 wg.astype(bf), u, cb)
    return out[:, :2 * N_CLS].reshape(B, SPAN_NUM, 2 * N_CLS)
